# Initial kernel scaffold; baseline (speedup 1.0000x reference)
import jax
import jax.numpy as jnp
from jax.experimental import pallas as pl


def kernel(x, edge_index_orders, edge_index_geographical, W_orders, b_orders, W_geographical, b_geographical):
    raise NotImplementedError("write your pallas kernel here")


# speedup vs baseline: 10.0629x; 10.0629x over previous
import functools

import jax
import jax.numpy as jnp
from jax import lax
from jax.experimental import pallas as pl
from jax.experimental.pallas import tpu as pltpu
from jax.experimental.pallas import tpu_sc as plsc

N = 10000
NPAD = 16
NP = N + NPAD
E = 320000
D = 128
NT = 16
C = 128
KB = 8
NBLK = 20
EP = NT * NBLK * KB * C
RPT = NP // NT


def _sc_mesh():
    return plsc.VectorSubcoreMesh(core_axis_name="c", subcore_axis_name="s")


def _edge_loop(ei_hbm, s, idx_src, idx_dst, chunk_fn):
    def b_body(b, carry):
        pltpu.sync_copy(ei_hbm.at[0, s, b], idx_src)
        pltpu.sync_copy(ei_hbm.at[1, s, b], idx_dst)

        def j_body(jj, carry2):
            chunk_fn(idx_src.at[jj], idx_dst.at[jj])
            return carry2

        lax.fori_loop(0, KB, j_body, 0)
        return carry

    lax.fori_loop(0, NBLK, b_body, 0)


def _deg_call(ei_o, ei_g, z128, ones128):
    @functools.partial(
        pl.kernel,
        out_type=jax.ShapeDtypeStruct((2, 2, NT, RPT, D), jnp.float32),
        mesh=_sc_mesh(),
        scratch_types=[
            pltpu.VMEM((KB, C), jnp.int32),
            pltpu.VMEM((KB, C), jnp.int32),
            pltpu.VMEM((C, D), jnp.float32),
            pltpu.VMEM_SHARED((NP, D), jnp.float32),
        ],
    )
    def deg(ei_o_hbm, ei_g_hbm, z128_hbm, ones_hbm, cnt_out,
            idx_src, idx_dst, ones_v, cnt_sh):
        c = lax.axis_index("c")
        s = lax.axis_index("s")
        row0 = s * RPT
        pltpu.sync_copy(ones_hbm, ones_v)

        for side in (0, 1):
            pltpu.sync_copy(z128_hbm, cnt_sh.at[pl.ds(row0, RPT)])
            plsc.subcore_barrier()

            def chunk(src_row, dst_row, _side=side):
                row = src_row if _side == 0 else dst_row
                pltpu.sync_copy(ones_v, cnt_sh.at[row], add=True)

            @pl.when(c == 0)
            def _():
                _edge_loop(ei_o_hbm, s, idx_src, idx_dst, chunk)

            @pl.when(c == 1)
            def _():
                _edge_loop(ei_g_hbm, s, idx_src, idx_dst, chunk)

            plsc.subcore_barrier()

            @pl.when(c == 0)
            def _():
                pltpu.sync_copy(cnt_sh.at[pl.ds(row0, RPT)],
                                cnt_out.at[0, side, s])

            @pl.when(c == 1)
            def _():
                pltpu.sync_copy(cnt_sh.at[pl.ds(row0, RPT)],
                                cnt_out.at[1, side, s])

            plsc.subcore_barrier()

    return deg(ei_o, ei_g, z128, ones128)


def _mm_body(x_ref, wo_ref, wg_ref, co_ref, cg_ref, yo_ref, yg_ref):
    xb = x_ref[...]
    ns_o = lax.rsqrt(jnp.maximum(co_ref[...], 1.0))
    ns_g = lax.rsqrt(jnp.maximum(cg_ref[...], 1.0))
    yo_ref[...] = jnp.dot(xb, wo_ref[...],
                          preferred_element_type=jnp.float32) * ns_o
    yg_ref[...] = jnp.dot(xb, wg_ref[...],
                          preferred_element_type=jnp.float32) * ns_g


def _mm_call(x, Wo, Wg, cs_o, cs_g):
    blk = 1000
    return pl.pallas_call(
        _mm_body,
        grid=(N // blk,),
        in_specs=[
            pl.BlockSpec((blk, D), lambda i: (i, 0)),
            pl.BlockSpec((D, D), lambda i: (0, 0)),
            pl.BlockSpec((D, D), lambda i: (0, 0)),
            pl.BlockSpec((blk, 1), lambda i: (i, 0)),
            pl.BlockSpec((blk, 1), lambda i: (i, 0)),
        ],
        out_specs=[
            pl.BlockSpec((blk, D), lambda i: (i, 0)),
            pl.BlockSpec((blk, D), lambda i: (i, 0)),
        ],
        out_shape=[
            jax.ShapeDtypeStruct((N, D), jnp.float32),
            jax.ShapeDtypeStruct((N, D), jnp.float32),
        ],
    )(x, Wo, Wg, cs_o, cs_g)


def _main_call(y_o, y_g, ei_o, ei_g, z128):
    @functools.partial(
        pl.kernel,
        out_type=[
            jax.ShapeDtypeStruct((NT, RPT, D), jnp.float32),
            jax.ShapeDtypeStruct((NT, RPT, D), jnp.float32),
        ],
        mesh=_sc_mesh(),
        scratch_types=[
            pltpu.VMEM((KB, C), jnp.int32),
            pltpu.VMEM((KB, C), jnp.int32),
            pltpu.VMEM((C, D), jnp.float32),
            pltpu.VMEM_SHARED((NP, D), jnp.float32),
            pltpu.SemaphoreType.DMA,
        ],
    )
    def main(y_o_hbm, y_g_hbm, ei_o_hbm, ei_g_hbm, z128_hbm,
             agg_o_out, agg_g_out, idx_src, idx_dst, rows, agg_sh, sem):
        c = lax.axis_index("c")
        s = lax.axis_index("s")
        row0 = s * RPT
        pltpu.sync_copy(z128_hbm, agg_sh.at[pl.ds(row0, RPT)])
        plsc.subcore_barrier()

        def chunk(y_hbm):
            def go(src_row, dst_row):
                pltpu.async_copy(y_hbm.at[src_row], rows, sem).wait()
                pltpu.sync_copy(rows, agg_sh.at[dst_row], add=True)
            return go

        @pl.when(c == 0)
        def _():
            _edge_loop(ei_o_hbm, s, idx_src, idx_dst, chunk(y_o_hbm))

        @pl.when(c == 1)
        def _():
            _edge_loop(ei_g_hbm, s, idx_src, idx_dst, chunk(y_g_hbm))

        plsc.subcore_barrier()

        @pl.when(c == 0)
        def _():
            pltpu.sync_copy(agg_sh.at[pl.ds(row0, RPT)], agg_o_out.at[s])

        @pl.when(c == 1)
        def _():
            pltpu.sync_copy(agg_sh.at[pl.ds(row0, RPT)], agg_g_out.at[s])

    return main(y_o, y_g, ei_o, ei_g, z128)


def _fin_body(ao_ref, ag_ref, cdo_ref, cdg_ref, bo_ref, bg_ref, out_ref):
    nd_o = lax.rsqrt(jnp.maximum(cdo_ref[...], 1.0))
    nd_g = lax.rsqrt(jnp.maximum(cdg_ref[...], 1.0))
    out_ref[...] = (ao_ref[...] * nd_o + ag_ref[...] * nd_g
                    + bo_ref[...] + bg_ref[...])


def _fin_call(agg_o, agg_g, cd_o, cd_g, bo, bg):
    blk = 1000
    return pl.pallas_call(
        _fin_body,
        grid=(N // blk,),
        in_specs=[
            pl.BlockSpec((blk, D), lambda i: (i, 0)),
            pl.BlockSpec((blk, D), lambda i: (i, 0)),
            pl.BlockSpec((blk, 1), lambda i: (i, 0)),
            pl.BlockSpec((blk, 1), lambda i: (i, 0)),
            pl.BlockSpec((1, D), lambda i: (0, 0)),
            pl.BlockSpec((1, D), lambda i: (0, 0)),
        ],
        out_specs=pl.BlockSpec((blk, D), lambda i: (i, 0)),
        out_shape=jax.ShapeDtypeStruct((N, D), jnp.float32),
    )(agg_o, agg_g, cd_o, cd_g, bo, bg)


def kernel(x, edge_index_orders, edge_index_geographical,
           W_orders, b_orders, W_geographical, b_geographical):
    pad = (jnp.arange(EP - E, dtype=jnp.int32) % NPAD) + N
    pad2 = jnp.stack([pad, pad])

    def prep(ei):
        return jnp.concatenate([ei, pad2], axis=1).reshape(2, NT, NBLK, KB, C)

    ei_o = prep(edge_index_orders)
    ei_g = prep(edge_index_geographical)
    ones128 = jnp.ones((C, D), jnp.float32)
    z128 = jnp.zeros((RPT, D), jnp.float32)

    cnt = _deg_call(ei_o, ei_g, z128, ones128).reshape(2, 2, NP, D)
    cs_o = cnt[0, 0, :N, 0:1]
    cd_o = cnt[0, 1, :N, 0:1]
    cs_g = cnt[1, 0, :N, 0:1]
    cd_g = cnt[1, 1, :N, 0:1]

    y_o, y_g = _mm_call(x, W_orders, W_geographical, cs_o, cs_g)
    ypad = jnp.zeros((NPAD, D), jnp.float32)
    y_o = jnp.concatenate([y_o, ypad], axis=0)
    y_g = jnp.concatenate([y_g, ypad], axis=0)

    agg_o, agg_g = _main_call(y_o, y_g, ei_o, ei_g, z128)
    agg_o = agg_o.reshape(NP, D)[:N]
    agg_g = agg_g.reshape(NP, D)[:N]
    return _fin_call(agg_o, agg_g, cd_o, cd_g,
                     b_orders.reshape(1, D), b_geographical.reshape(1, D))


# speedup vs baseline: 19.5300x; 1.9408x over previous
import functools

import jax
import jax.numpy as jnp
from jax import lax
from jax.experimental import pallas as pl
from jax.experimental.pallas import tpu as pltpu
from jax.experimental.pallas import tpu_sc as plsc

N = 10000
NPAD = 16
NP = N + NPAD
E = 320000
D = 128
NT = 16
C = 128
KB = 8
NBLK = 20
EP = NT * NBLK * KB * C
RPT = NP // NT


def _sc_mesh():
    return plsc.VectorSubcoreMesh(core_axis_name="c", subcore_axis_name="s")


def _edge_loop(ei_hbm, s, idx_src, idx_dst, chunk_fn):
    def b_body(b, carry):
        pltpu.sync_copy(ei_hbm.at[0, s, b], idx_src)
        pltpu.sync_copy(ei_hbm.at[1, s, b], idx_dst)

        def j_body(jj, carry2):
            chunk_fn(idx_src.at[jj], idx_dst.at[jj])
            return carry2

        lax.fori_loop(0, KB, j_body, 0)
        return carry

    lax.fori_loop(0, NBLK, b_body, 0)


WIN = 640
NPX = NT * WIN
WR = WIN // 128


def _deg_call(ei_o, ei_g, z1, ones1):
    @functools.partial(
        pl.kernel,
        out_type=jax.ShapeDtypeStruct((2, 2, NT, WR, 128), jnp.float32),
        mesh=_sc_mesh(),
        scratch_types=[
            pltpu.VMEM((KB, C), jnp.int32),
            pltpu.VMEM((KB, C), jnp.int32),
            pltpu.VMEM((C,), jnp.float32),
            pltpu.VMEM((WIN,), jnp.float32),
            pltpu.VMEM((WR, 128), jnp.float32),
            pltpu.VMEM_SHARED((NPX,), jnp.float32),
            pltpu.VMEM_SHARED((NPX,), jnp.float32),
        ],
    )
    def deg(ei_o_hbm, ei_g_hbm, z1_hbm, ones_hbm, cnt_out,
            idx_src, idx_dst, ones_v, buf1, buf2, cs_sh, cd_sh):
        c = lax.axis_index("c")
        s = lax.axis_index("s")
        w0 = s * WIN
        pltpu.sync_copy(ones_hbm, ones_v)
        pltpu.sync_copy(z1_hbm, cs_sh.at[pl.ds(w0, WIN)])
        pltpu.sync_copy(z1_hbm, cd_sh.at[pl.ds(w0, WIN)])
        plsc.subcore_barrier()

        def chunk(src_row, dst_row):
            pltpu.sync_copy(ones_v, cs_sh.at[src_row], add=True)
            pltpu.sync_copy(ones_v, cd_sh.at[dst_row], add=True)

        @pl.when(c == 0)
        def _():
            _edge_loop(ei_o_hbm, s, idx_src, idx_dst, chunk)

        @pl.when(c == 1)
        def _():
            _edge_loop(ei_g_hbm, s, idx_src, idx_dst, chunk)

        plsc.subcore_barrier()

        def emit(sh, m, side):
            pltpu.sync_copy(sh.at[pl.ds(w0, WIN)], buf1)
            for k in range(WIN // 16):
                buf2[k // 8, pl.ds((k % 8) * 16, 16)] = buf1[pl.ds(k * 16, 16)]
            pltpu.sync_copy(buf2, cnt_out.at[m, side, s])

        @pl.when(c == 0)
        def _():
            emit(cs_sh, 0, 0)
            emit(cd_sh, 0, 1)

        @pl.when(c == 1)
        def _():
            emit(cs_sh, 1, 0)
            emit(cd_sh, 1, 1)

    return deg(ei_o, ei_g, z1, ones1)


def _mm_body(x_ref, wo_ref, wg_ref, co_ref, cg_ref, yo_ref, yg_ref):
    xb = x_ref[...]
    ns_o = lax.rsqrt(jnp.maximum(co_ref[...], 1.0))
    ns_g = lax.rsqrt(jnp.maximum(cg_ref[...], 1.0))
    yo_ref[...] = jnp.dot(xb, wo_ref[...],
                          preferred_element_type=jnp.float32) * ns_o
    yg_ref[...] = jnp.dot(xb, wg_ref[...],
                          preferred_element_type=jnp.float32) * ns_g


def _mm_call(x, Wo, Wg, cs_o, cs_g):
    blk = 1000
    return pl.pallas_call(
        _mm_body,
        grid=(N // blk,),
        in_specs=[
            pl.BlockSpec((blk, D), lambda i: (i, 0)),
            pl.BlockSpec((D, D), lambda i: (0, 0)),
            pl.BlockSpec((D, D), lambda i: (0, 0)),
            pl.BlockSpec((blk, 1), lambda i: (i, 0)),
            pl.BlockSpec((blk, 1), lambda i: (i, 0)),
        ],
        out_specs=[
            pl.BlockSpec((blk, D), lambda i: (i, 0)),
            pl.BlockSpec((blk, D), lambda i: (i, 0)),
        ],
        out_shape=[
            jax.ShapeDtypeStruct((N, D), jnp.float32),
            jax.ShapeDtypeStruct((N, D), jnp.float32),
        ],
    )(x, Wo, Wg, cs_o, cs_g)


def _main_call(y_o, y_g, ei_o, ei_g, z128):
    @functools.partial(
        pl.kernel,
        out_type=[
            jax.ShapeDtypeStruct((NT, RPT, D), jnp.float32),
            jax.ShapeDtypeStruct((NT, RPT, D), jnp.float32),
        ],
        mesh=_sc_mesh(),
        scratch_types=[
            pltpu.VMEM((KB, C), jnp.int32),
            pltpu.VMEM((KB, C), jnp.int32),
            pltpu.VMEM((C, D), jnp.float32),
            pltpu.VMEM((C, D), jnp.float32),
            pltpu.VMEM_SHARED((NP, D), jnp.float32),
            pltpu.SemaphoreType.DMA,
            pltpu.SemaphoreType.DMA,
        ],
    )
    def main(y_o_hbm, y_g_hbm, ei_o_hbm, ei_g_hbm, z128_hbm,
             agg_o_out, agg_g_out, idx_src, idx_dst, rowsA, rowsB, agg_sh,
             semA, semB):
        c = lax.axis_index("c")
        s = lax.axis_index("s")
        row0 = s * RPT
        pltpu.sync_copy(z128_hbm, agg_sh.at[pl.ds(row0, RPT)])
        plsc.subcore_barrier()

        bufs = (rowsA, rowsB)
        sems = (semA, semB)

        def edge_loop(y_hbm, ei_hbm):
            def b_body(b, carry):
                pltpu.sync_copy(ei_hbm.at[0, s, b], idx_src)
                pltpu.sync_copy(ei_hbm.at[1, s, b], idx_dst)
                d = pltpu.async_copy(y_hbm.at[idx_src.at[0]], bufs[0],
                                     sems[0])
                for jj in range(KB):
                    if jj + 1 < KB:
                        d_next = pltpu.async_copy(
                            y_hbm.at[idx_src.at[jj + 1]],
                            bufs[(jj + 1) % 2], sems[(jj + 1) % 2])
                    d.wait()
                    pltpu.sync_copy(bufs[jj % 2],
                                    agg_sh.at[idx_dst.at[jj]], add=True)
                    if jj + 1 < KB:
                        d = d_next
                return carry

            lax.fori_loop(0, NBLK, b_body, 0)

        @pl.when(c == 0)
        def _():
            edge_loop(y_o_hbm, ei_o_hbm)

        @pl.when(c == 1)
        def _():
            edge_loop(y_g_hbm, ei_g_hbm)

        plsc.subcore_barrier()

        @pl.when(c == 0)
        def _():
            pltpu.sync_copy(agg_sh.at[pl.ds(row0, RPT)], agg_o_out.at[s])

        @pl.when(c == 1)
        def _():
            pltpu.sync_copy(agg_sh.at[pl.ds(row0, RPT)], agg_g_out.at[s])

    return main(y_o, y_g, ei_o, ei_g, z128)


def _fin_body(ao_ref, ag_ref, cdo_ref, cdg_ref, bo_ref, bg_ref, out_ref):
    nd_o = lax.rsqrt(jnp.maximum(cdo_ref[...], 1.0))
    nd_g = lax.rsqrt(jnp.maximum(cdg_ref[...], 1.0))
    out_ref[...] = (ao_ref[...] * nd_o + ag_ref[...] * nd_g
                    + bo_ref[...] + bg_ref[...])


def _fin_call(agg_o, agg_g, cd_o, cd_g, bo, bg):
    blk = 1000
    return pl.pallas_call(
        _fin_body,
        grid=(N // blk,),
        in_specs=[
            pl.BlockSpec((blk, D), lambda i: (i, 0)),
            pl.BlockSpec((blk, D), lambda i: (i, 0)),
            pl.BlockSpec((blk, 1), lambda i: (i, 0)),
            pl.BlockSpec((blk, 1), lambda i: (i, 0)),
            pl.BlockSpec((1, D), lambda i: (0, 0)),
            pl.BlockSpec((1, D), lambda i: (0, 0)),
        ],
        out_specs=pl.BlockSpec((blk, D), lambda i: (i, 0)),
        out_shape=jax.ShapeDtypeStruct((N, D), jnp.float32),
    )(agg_o, agg_g, cd_o, cd_g, bo, bg)


def kernel(x, edge_index_orders, edge_index_geographical,
           W_orders, b_orders, W_geographical, b_geographical):
    pad = (jnp.arange(EP - E, dtype=jnp.int32) % NPAD) + N
    pad2 = jnp.stack([pad, pad])

    def prep(ei):
        return jnp.concatenate([ei, pad2], axis=1).reshape(2, NT, NBLK, KB, C)

    ei_o = prep(edge_index_orders)
    ei_g = prep(edge_index_geographical)
    ones1 = jnp.ones((C,), jnp.float32)
    z1 = jnp.zeros((WIN,), jnp.float32)
    z128 = jnp.zeros((RPT, D), jnp.float32)

    cnt = _deg_call(ei_o, ei_g, z1, ones1).reshape(2, 2, NPX)
    cs_o = cnt[0, 0, :N, None]
    cd_o = cnt[0, 1, :N, None]
    cs_g = cnt[1, 0, :N, None]
    cd_g = cnt[1, 1, :N, None]

    y_o, y_g = _mm_call(x, W_orders, W_geographical, cs_o, cs_g)
    ypad = jnp.zeros((NPAD, D), jnp.float32)
    y_o = jnp.concatenate([y_o, ypad], axis=0)
    y_g = jnp.concatenate([y_g, ypad], axis=0)

    agg_o, agg_g = _main_call(y_o, y_g, ei_o, ei_g, z128)
    agg_o = agg_o.reshape(NP, D)[:N]
    agg_g = agg_g.reshape(NP, D)[:N]
    return _fin_call(agg_o, agg_g, cd_o, cd_g,
                     b_orders.reshape(1, D), b_geographical.reshape(1, D))


# speedup vs baseline: 22.5294x; 1.1536x over previous
import functools

import jax
import jax.numpy as jnp
from jax import lax
from jax.experimental import pallas as pl
from jax.experimental.pallas import tpu as pltpu
from jax.experimental.pallas import tpu_sc as plsc

N = 10000
NPAD = 16
NP = N + NPAD
E = 320000
D = 128
NT = 16
C = 128
KB = 16
NBLK = 10
EP = NT * NBLK * KB * C
RPT = NP // NT


def _sc_mesh():
    return plsc.VectorSubcoreMesh(core_axis_name="c", subcore_axis_name="s")


WIN = 640
NPX = NT * WIN
WR = WIN // 128


def _deg_call(ei_o, ei_g, z1, ones1):
    @functools.partial(
        pl.kernel,
        out_type=jax.ShapeDtypeStruct((2, 2, NT, WR, 128), jnp.float32),
        mesh=_sc_mesh(),
        scratch_types=[
            pltpu.VMEM((KB, C), jnp.int32),
            pltpu.VMEM((KB, C), jnp.int32),
            pltpu.VMEM((C,), jnp.float32),
            pltpu.VMEM((WIN,), jnp.float32),
            pltpu.VMEM((WR, 128), jnp.float32),
            pltpu.VMEM_SHARED((NPX,), jnp.float32),
            pltpu.VMEM_SHARED((NPX,), jnp.float32),
            pltpu.SemaphoreType.DMA,
            pltpu.SemaphoreType.DMA,
            pltpu.SemaphoreType.DMA,
            pltpu.SemaphoreType.DMA,
        ],
    )
    def deg(ei_o_hbm, ei_g_hbm, z1_hbm, ones_hbm, cnt_out,
            idx_src, idx_dst, ones_v, buf1, buf2, cs_sh, cd_sh,
            sA0, sA1, sB0, sB1):
        sems = ((sA0, sA1), (sB0, sB1))
        c = lax.axis_index("c")
        s = lax.axis_index("s")
        w0 = s * WIN
        pltpu.sync_copy(ones_hbm, ones_v)
        pltpu.sync_copy(z1_hbm, cs_sh.at[pl.ds(w0, WIN)])
        pltpu.sync_copy(z1_hbm, cd_sh.at[pl.ds(w0, WIN)])
        plsc.subcore_barrier()

        def edge_loop(ei_hbm):
            def b_body(b, carry):
                pltpu.sync_copy(ei_hbm.at[0, s, b], idx_src)
                pltpu.sync_copy(ei_hbm.at[1, s, b], idx_dst)
                for jj in range(KB):
                    da = pltpu.async_copy(ones_v, cs_sh.at[idx_src.at[jj]],
                                          sems[0][0], add=True)
                    db = pltpu.async_copy(ones_v, cd_sh.at[idx_dst.at[jj]],
                                          sems[0][1], add=True)
                    da.wait()
                    db.wait()
                return carry

            lax.fori_loop(0, NBLK, b_body, 0)

        @pl.when(c == 0)
        def _():
            edge_loop(ei_o_hbm)

        @pl.when(c == 1)
        def _():
            edge_loop(ei_g_hbm)

        plsc.subcore_barrier()

        def emit(sh, m, side):
            pltpu.sync_copy(sh.at[pl.ds(w0, WIN)], buf1)
            for k in range(WIN // 16):
                buf2[k // 8, pl.ds((k % 8) * 16, 16)] = buf1[pl.ds(k * 16, 16)]
            pltpu.sync_copy(buf2, cnt_out.at[m, side, s])

        @pl.when(c == 0)
        def _():
            emit(cs_sh, 0, 0)
            emit(cd_sh, 0, 1)

        @pl.when(c == 1)
        def _():
            emit(cs_sh, 1, 0)
            emit(cd_sh, 1, 1)

    return deg(ei_o, ei_g, z1, ones1)


def _mm_body(x_ref, wo_ref, wg_ref, co_ref, cg_ref, yo_ref, yg_ref):
    xb = x_ref[...]
    ns_o = lax.rsqrt(jnp.maximum(co_ref[...], 1.0))
    ns_g = lax.rsqrt(jnp.maximum(cg_ref[...], 1.0))
    yo_ref[...] = jnp.dot(xb, wo_ref[...],
                          preferred_element_type=jnp.float32) * ns_o
    yg_ref[...] = jnp.dot(xb, wg_ref[...],
                          preferred_element_type=jnp.float32) * ns_g


def _mm_call(x, Wo, Wg, cs_o, cs_g):
    blk = 1000
    return pl.pallas_call(
        _mm_body,
        grid=(N // blk,),
        in_specs=[
            pl.BlockSpec((blk, D), lambda i: (i, 0)),
            pl.BlockSpec((D, D), lambda i: (0, 0)),
            pl.BlockSpec((D, D), lambda i: (0, 0)),
            pl.BlockSpec((blk, 1), lambda i: (i, 0)),
            pl.BlockSpec((blk, 1), lambda i: (i, 0)),
        ],
        out_specs=[
            pl.BlockSpec((blk, D), lambda i: (i, 0)),
            pl.BlockSpec((blk, D), lambda i: (i, 0)),
        ],
        out_shape=[
            jax.ShapeDtypeStruct((N, D), jnp.float32),
            jax.ShapeDtypeStruct((N, D), jnp.float32),
        ],
    )(x, Wo, Wg, cs_o, cs_g)


def _main_call(y_o, y_g, ei_o, ei_g, z128):
    @functools.partial(
        pl.kernel,
        out_type=[
            jax.ShapeDtypeStruct((NT, RPT, D), jnp.float32),
            jax.ShapeDtypeStruct((NT, RPT, D), jnp.float32),
        ],
        mesh=_sc_mesh(),
        scratch_types=[
            pltpu.VMEM((KB, C), jnp.int32),
            pltpu.VMEM((KB, C), jnp.int32),
            pltpu.VMEM((2, C, D), jnp.float32),
            pltpu.VMEM_SHARED((NP, D), jnp.float32),
            pltpu.SemaphoreType.DMA,
            pltpu.SemaphoreType.DMA,
            pltpu.SemaphoreType.DMA,
            pltpu.SemaphoreType.DMA,
        ],
    )
    def main(y_o_hbm, y_g_hbm, ei_o_hbm, ei_g_hbm, z128_hbm,
             agg_o_out, agg_g_out, idx_src, idx_dst, rows, agg_sh,
             g0, g1, p0, p1):
        c = lax.axis_index("c")
        s = lax.axis_index("s")
        row0 = s * RPT
        pltpu.sync_copy(z128_hbm, agg_sh.at[pl.ds(row0, RPT)])
        plsc.subcore_barrier()

        gsems = (g0, g1)
        psems = (p0, p1)
        NBUF = 2
        LAG = 1

        def edge_loop(y_hbm, ei_hbm):
            def b_body(b, carry):
                pltpu.sync_copy(ei_hbm.at[0, s, b], idx_src)
                pltpu.sync_copy(ei_hbm.at[1, s, b], idx_dst)
                d = pltpu.async_copy(y_hbm.at[idx_src.at[0]], rows.at[0],
                                     gsems[0])
                for jj in range(KB):
                    if jj + 1 < KB:
                        d_next = pltpu.async_copy(
                            y_hbm.at[idx_src.at[jj + 1]],
                            rows.at[(jj + 1) % 2], gsems[(jj + 1) % 2])
                    d.wait()
                    pltpu.sync_copy(rows.at[jj % 2],
                                    agg_sh.at[idx_dst.at[jj]], add=True)
                    if jj + 1 < KB:
                        d = d_next
                return carry

            lax.fori_loop(0, NBLK, b_body, 0)

        @pl.when(c == 0)
        def _():
            edge_loop(y_o_hbm, ei_o_hbm)

        @pl.when(c == 1)
        def _():
            edge_loop(y_g_hbm, ei_g_hbm)

        plsc.subcore_barrier()

        @pl.when(c == 0)
        def _():
            pltpu.sync_copy(agg_sh.at[pl.ds(row0, RPT)], agg_o_out.at[s])

        @pl.when(c == 1)
        def _():
            pltpu.sync_copy(agg_sh.at[pl.ds(row0, RPT)], agg_g_out.at[s])

    return main(y_o, y_g, ei_o, ei_g, z128)


def _fin_body(ao_ref, ag_ref, cdo_ref, cdg_ref, bo_ref, bg_ref, out_ref):
    nd_o = lax.rsqrt(jnp.maximum(cdo_ref[...], 1.0))
    nd_g = lax.rsqrt(jnp.maximum(cdg_ref[...], 1.0))
    out_ref[...] = (ao_ref[...] * nd_o + ag_ref[...] * nd_g
                    + bo_ref[...] + bg_ref[...])


def _fin_call(agg_o, agg_g, cd_o, cd_g, bo, bg):
    blk = 1000
    return pl.pallas_call(
        _fin_body,
        grid=(N // blk,),
        in_specs=[
            pl.BlockSpec((blk, D), lambda i: (i, 0)),
            pl.BlockSpec((blk, D), lambda i: (i, 0)),
            pl.BlockSpec((blk, 1), lambda i: (i, 0)),
            pl.BlockSpec((blk, 1), lambda i: (i, 0)),
            pl.BlockSpec((1, D), lambda i: (0, 0)),
            pl.BlockSpec((1, D), lambda i: (0, 0)),
        ],
        out_specs=pl.BlockSpec((blk, D), lambda i: (i, 0)),
        out_shape=jax.ShapeDtypeStruct((N, D), jnp.float32),
    )(agg_o, agg_g, cd_o, cd_g, bo, bg)


def kernel(x, edge_index_orders, edge_index_geographical,
           W_orders, b_orders, W_geographical, b_geographical):
    ar = jnp.arange(EP - E, dtype=jnp.int32) % NPAD
    pad2 = jnp.stack([ar, ar + N])

    def prep(ei):
        return jnp.concatenate([ei, pad2], axis=1).reshape(2, NT, NBLK, KB, C)

    ei_o = prep(edge_index_orders)
    ei_g = prep(edge_index_geographical)
    ones1 = jnp.ones((C,), jnp.float32)
    z1 = jnp.zeros((WIN,), jnp.float32)
    z128 = jnp.zeros((RPT, D), jnp.float32)

    cnt = _deg_call(ei_o, ei_g, z1, ones1).reshape(2, 2, NPX)
    pc = jnp.zeros((N,), jnp.float32).at[:NPAD].set((EP - E) // NPAD)
    cs_o = (cnt[0, 0, :N] - pc)[:, None]
    cd_o = cnt[0, 1, :N, None]
    cs_g = (cnt[1, 0, :N] - pc)[:, None]
    cd_g = cnt[1, 1, :N, None]

    y_o, y_g = _mm_call(x, W_orders, W_geographical, cs_o, cs_g)
    agg_o, agg_g = _main_call(y_o, y_g, ei_o, ei_g, z128)
    return _fin_call(agg_o.reshape(NP, D), agg_g.reshape(NP, D), cd_o, cd_g,
                     b_orders.reshape(1, D), b_geographical.reshape(1, D))


# speedup vs baseline: 23.1995x; 1.0297x over previous
import functools

import jax
import jax.numpy as jnp
from jax import lax
from jax.experimental import pallas as pl
from jax.experimental.pallas import tpu as pltpu
from jax.experimental.pallas import tpu_sc as plsc

N = 10000
NPAD = 16
NP = N + NPAD
E = 320000
D = 128
NT = 16
C = 128
KB = 16
NBLK = 10
EP = NT * NBLK * KB * C
RPT = NP // NT


def _sc_mesh():
    return plsc.VectorSubcoreMesh(core_axis_name="c", subcore_axis_name="s")


WIN = 640
NPX = NT * WIN
WR = WIN // 128


def _deg_call(ei_o, ei_g, z1, ones1):
    @functools.partial(
        pl.kernel,
        out_type=jax.ShapeDtypeStruct((2, 2, NT, WR, 128), jnp.float32),
        mesh=_sc_mesh(),
        scratch_types=[
            pltpu.VMEM((KB, C), jnp.int32),
            pltpu.VMEM((KB, C), jnp.int32),
            pltpu.VMEM((C,), jnp.float32),
            pltpu.VMEM((WIN,), jnp.float32),
            pltpu.VMEM((WR, 128), jnp.float32),
            pltpu.VMEM_SHARED((NPX,), jnp.float32),
            pltpu.VMEM_SHARED((NPX,), jnp.float32),
            pltpu.SemaphoreType.DMA,
            pltpu.SemaphoreType.DMA,
            pltpu.SemaphoreType.DMA,
            pltpu.SemaphoreType.DMA,
        ],
    )
    def deg(ei_o_hbm, ei_g_hbm, z1_hbm, ones_hbm, cnt_out,
            idx_src, idx_dst, ones_v, buf1, buf2, cs_sh, cd_sh,
            sA0, sA1, sB0, sB1):
        sems = ((sA0, sA1), (sB0, sB1))
        c = lax.axis_index("c")
        s = lax.axis_index("s")
        w0 = s * WIN
        pltpu.sync_copy(ones_hbm, ones_v)
        pltpu.sync_copy(z1_hbm, cs_sh.at[pl.ds(w0, WIN)])
        pltpu.sync_copy(z1_hbm, cd_sh.at[pl.ds(w0, WIN)])
        plsc.subcore_barrier()

        def edge_loop(ei_hbm):
            def b_body(b, carry):
                pltpu.sync_copy(ei_hbm.at[0, s, b], idx_src)
                pltpu.sync_copy(ei_hbm.at[1, s, b], idx_dst)
                prev = None
                for jj in range(KB):
                    cur = (
                        pltpu.async_copy(ones_v, cs_sh.at[idx_src.at[jj]],
                                         sems[jj % 2][0], add=True),
                        pltpu.async_copy(ones_v, cd_sh.at[idx_dst.at[jj]],
                                         sems[jj % 2][1], add=True),
                    )
                    if prev is not None:
                        prev[0].wait()
                        prev[1].wait()
                    prev = cur
                prev[0].wait()
                prev[1].wait()
                return carry

            lax.fori_loop(0, NBLK, b_body, 0)

        @pl.when(c == 0)
        def _():
            edge_loop(ei_o_hbm)

        @pl.when(c == 1)
        def _():
            edge_loop(ei_g_hbm)

        plsc.subcore_barrier()

        def emit(sh, m, side):
            pltpu.sync_copy(sh.at[pl.ds(w0, WIN)], buf1)
            for k in range(WIN // 16):
                buf2[k // 8, pl.ds((k % 8) * 16, 16)] = buf1[pl.ds(k * 16, 16)]
            pltpu.sync_copy(buf2, cnt_out.at[m, side, s])

        @pl.when(c == 0)
        def _():
            emit(cs_sh, 0, 0)
            emit(cd_sh, 0, 1)

        @pl.when(c == 1)
        def _():
            emit(cs_sh, 1, 0)
            emit(cd_sh, 1, 1)

    return deg(ei_o, ei_g, z1, ones1)


def _mm_body(x_ref, wo_ref, wg_ref, co_ref, cg_ref, yo_ref, yg_ref):
    xb = x_ref[...]
    ns_o = lax.rsqrt(jnp.maximum(co_ref[...], 1.0))
    ns_g = lax.rsqrt(jnp.maximum(cg_ref[...], 1.0))
    yo_ref[...] = jnp.dot(xb, wo_ref[...],
                          preferred_element_type=jnp.float32) * ns_o
    yg_ref[...] = jnp.dot(xb, wg_ref[...],
                          preferred_element_type=jnp.float32) * ns_g


def _mm_call(x, Wo, Wg, cs_o, cs_g):
    blk = 1000
    return pl.pallas_call(
        _mm_body,
        grid=(N // blk,),
        in_specs=[
            pl.BlockSpec((blk, D), lambda i: (i, 0)),
            pl.BlockSpec((D, D), lambda i: (0, 0)),
            pl.BlockSpec((D, D), lambda i: (0, 0)),
            pl.BlockSpec((blk, 1), lambda i: (i, 0)),
            pl.BlockSpec((blk, 1), lambda i: (i, 0)),
        ],
        out_specs=[
            pl.BlockSpec((blk, D), lambda i: (i, 0)),
            pl.BlockSpec((blk, D), lambda i: (i, 0)),
        ],
        out_shape=[
            jax.ShapeDtypeStruct((N, D), jnp.float32),
            jax.ShapeDtypeStruct((N, D), jnp.float32),
        ],
    )(x, Wo, Wg, cs_o, cs_g)


def _main_call(y_o, y_g, ei_o, ei_g, z128):
    @functools.partial(
        pl.kernel,
        out_type=[
            jax.ShapeDtypeStruct((NT, RPT, D), jnp.float32),
            jax.ShapeDtypeStruct((NT, RPT, D), jnp.float32),
        ],
        mesh=_sc_mesh(),
        scratch_types=[
            pltpu.VMEM((KB, C), jnp.int32),
            pltpu.VMEM((KB, C), jnp.int32),
            pltpu.VMEM((2, C, D), jnp.float32),
            pltpu.VMEM_SHARED((NP, D), jnp.float32),
            pltpu.SemaphoreType.DMA,
            pltpu.SemaphoreType.DMA,
            pltpu.SemaphoreType.DMA,
            pltpu.SemaphoreType.DMA,
        ],
    )
    def main(y_o_hbm, y_g_hbm, ei_o_hbm, ei_g_hbm, z128_hbm,
             agg_o_out, agg_g_out, idx_src, idx_dst, rows, agg_sh,
             g0, g1, p0, p1):
        c = lax.axis_index("c")
        s = lax.axis_index("s")
        row0 = s * RPT
        pltpu.sync_copy(z128_hbm, agg_sh.at[pl.ds(row0, RPT)])
        plsc.subcore_barrier()

        gsems = (g0, g1)
        psems = (p0, p1)
        NBUF = 2
        LAG = 1

        def edge_loop(y_hbm, ei_hbm):
            def b_body(b, carry):
                pltpu.sync_copy(ei_hbm.at[0, s, b], idx_src)
                pltpu.sync_copy(ei_hbm.at[1, s, b], idx_dst)
                dg = [None] * KB
                dsc = [None] * KB
                for t in range(KB + LAG):
                    jj = t
                    if jj < KB:
                        bb = jj % NBUF
                        if jj >= NBUF:
                            dsc[jj - NBUF].wait()
                        dg[jj] = pltpu.async_copy(
                            y_hbm.at[idx_src.at[jj]], rows.at[bb],
                            gsems[bb])
                    kk = t - LAG
                    if 0 <= kk < KB:
                        bb = kk % NBUF
                        dg[kk].wait()
                        dsc[kk] = pltpu.async_copy(
                            rows.at[bb], agg_sh.at[idx_dst.at[kk]],
                            psems[bb], add=True)
                for kk in range(KB - NBUF, KB):
                    dsc[kk].wait()
                return carry

            lax.fori_loop(0, NBLK, b_body, 0)

        @pl.when(c == 0)
        def _():
            edge_loop(y_o_hbm, ei_o_hbm)

        @pl.when(c == 1)
        def _():
            edge_loop(y_g_hbm, ei_g_hbm)

        plsc.subcore_barrier()

        @pl.when(c == 0)
        def _():
            pltpu.sync_copy(agg_sh.at[pl.ds(row0, RPT)], agg_o_out.at[s])

        @pl.when(c == 1)
        def _():
            pltpu.sync_copy(agg_sh.at[pl.ds(row0, RPT)], agg_g_out.at[s])

    return main(y_o, y_g, ei_o, ei_g, z128)


def _fin_body(ao_ref, ag_ref, cdo_ref, cdg_ref, bo_ref, bg_ref, out_ref):
    nd_o = lax.rsqrt(jnp.maximum(cdo_ref[...], 1.0))
    nd_g = lax.rsqrt(jnp.maximum(cdg_ref[...], 1.0))
    out_ref[...] = (ao_ref[...] * nd_o + ag_ref[...] * nd_g
                    + bo_ref[...] + bg_ref[...])


def _fin_call(agg_o, agg_g, cd_o, cd_g, bo, bg):
    blk = 1000
    return pl.pallas_call(
        _fin_body,
        grid=(N // blk,),
        in_specs=[
            pl.BlockSpec((blk, D), lambda i: (i, 0)),
            pl.BlockSpec((blk, D), lambda i: (i, 0)),
            pl.BlockSpec((blk, 1), lambda i: (i, 0)),
            pl.BlockSpec((blk, 1), lambda i: (i, 0)),
            pl.BlockSpec((1, D), lambda i: (0, 0)),
            pl.BlockSpec((1, D), lambda i: (0, 0)),
        ],
        out_specs=pl.BlockSpec((blk, D), lambda i: (i, 0)),
        out_shape=jax.ShapeDtypeStruct((N, D), jnp.float32),
    )(agg_o, agg_g, cd_o, cd_g, bo, bg)


def kernel(x, edge_index_orders, edge_index_geographical,
           W_orders, b_orders, W_geographical, b_geographical):
    ar = jnp.arange(EP - E, dtype=jnp.int32) % NPAD
    pad2 = jnp.stack([ar, ar + N])

    def prep(ei):
        return jnp.concatenate([ei, pad2], axis=1).reshape(2, NT, NBLK, KB, C)

    ei_o = prep(edge_index_orders)
    ei_g = prep(edge_index_geographical)
    ones1 = jnp.ones((C,), jnp.float32)
    z1 = jnp.zeros((WIN,), jnp.float32)
    z128 = jnp.zeros((RPT, D), jnp.float32)

    cnt = _deg_call(ei_o, ei_g, z1, ones1).reshape(2, 2, NPX)
    pc = jnp.zeros((N,), jnp.float32).at[:NPAD].set((EP - E) // NPAD)
    cs_o = (cnt[0, 0, :N] - pc)[:, None]
    cd_o = cnt[0, 1, :N, None]
    cs_g = (cnt[1, 0, :N] - pc)[:, None]
    cd_g = cnt[1, 1, :N, None]

    y_o, y_g = _mm_call(x, W_orders, W_geographical, cs_o, cs_g)
    agg_o, agg_g = _main_call(y_o, y_g, ei_o, ei_g, z128)
    return _fin_call(agg_o.reshape(NP, D), agg_g.reshape(NP, D), cd_o, cd_g,
                     b_orders.reshape(1, D), b_geographical.reshape(1, D))


# speedup vs baseline: 24.0086x; 1.0349x over previous
import functools

import jax
import jax.numpy as jnp
from jax import lax
from jax.experimental import pallas as pl
from jax.experimental.pallas import tpu as pltpu
from jax.experimental.pallas import tpu_sc as plsc

N = 10000
NPAD = 16
NP = N + NPAD
E = 320000
D = 128
NT = 16
C = 128
KB = 16
NBLK = 10
EP = NT * NBLK * KB * C
RPT = NP // NT


def _sc_mesh():
    return plsc.VectorSubcoreMesh(core_axis_name="c", subcore_axis_name="s")


WIN = 640
NPX = NT * WIN
WR = WIN // 128


def _deg_call(ei_o, ei_g, z1, ones1):
    @functools.partial(
        pl.kernel,
        out_type=jax.ShapeDtypeStruct((2, 2, NT, WR, 128), jnp.float32),
        mesh=_sc_mesh(),
        scratch_types=[
            pltpu.VMEM((KB, C), jnp.int32),
            pltpu.VMEM((KB, C), jnp.int32),
            pltpu.VMEM((C,), jnp.float32),
            pltpu.VMEM((WIN,), jnp.float32),
            pltpu.VMEM((WR, 128), jnp.float32),
            pltpu.VMEM_SHARED((NPX,), jnp.float32),
            pltpu.VMEM_SHARED((NPX,), jnp.float32),
            pltpu.SemaphoreType.DMA,
            pltpu.SemaphoreType.DMA,
            pltpu.SemaphoreType.DMA,
            pltpu.SemaphoreType.DMA,
        ],
    )
    def deg(ei_o_hbm, ei_g_hbm, z1_hbm, ones_hbm, cnt_out,
            idx_src, idx_dst, ones_v, buf1, buf2, cs_sh, cd_sh,
            sA0, sA1, sB0, sB1):
        sems = ((sA0, sA1), (sB0, sB1))
        c = lax.axis_index("c")
        s = lax.axis_index("s")
        w0 = s * WIN
        pltpu.sync_copy(ones_hbm, ones_v)
        pltpu.sync_copy(z1_hbm, cs_sh.at[pl.ds(w0, WIN)])
        pltpu.sync_copy(z1_hbm, cd_sh.at[pl.ds(w0, WIN)])
        plsc.subcore_barrier()

        def edge_loop(ei_hbm):
            def b_body(b, carry):
                pltpu.sync_copy(ei_hbm.at[0, s, b], idx_src)
                pltpu.sync_copy(ei_hbm.at[1, s, b], idx_dst)
                prev = None
                for jj in range(KB):
                    cur = (
                        pltpu.async_copy(ones_v, cs_sh.at[idx_src.at[jj]],
                                         sems[jj % 2][0], add=True),
                        pltpu.async_copy(ones_v, cd_sh.at[idx_dst.at[jj]],
                                         sems[jj % 2][1], add=True),
                    )
                    if prev is not None:
                        prev[0].wait()
                        prev[1].wait()
                    prev = cur
                prev[0].wait()
                prev[1].wait()
                return carry

            lax.fori_loop(0, NBLK, b_body, 0)

        @pl.when(c == 0)
        def _():
            edge_loop(ei_o_hbm)

        @pl.when(c == 1)
        def _():
            edge_loop(ei_g_hbm)

        plsc.subcore_barrier()

        def emit(sh, m, side):
            pltpu.sync_copy(sh.at[pl.ds(w0, WIN)], buf1)
            for k in range(WIN // 16):
                buf2[k // 8, pl.ds((k % 8) * 16, 16)] = buf1[pl.ds(k * 16, 16)]
            pltpu.sync_copy(buf2, cnt_out.at[m, side, s])

        @pl.when(c == 0)
        def _():
            emit(cs_sh, 0, 0)
            emit(cd_sh, 0, 1)

        @pl.when(c == 1)
        def _():
            emit(cs_sh, 1, 0)
            emit(cd_sh, 1, 1)

    return deg(ei_o, ei_g, z1, ones1)


def _mm_body(x_ref, wo_ref, wg_ref, co_ref, cg_ref, yo_ref, yg_ref):
    xb = x_ref[...]
    ns_o = lax.rsqrt(jnp.maximum(co_ref[...], 1.0))
    ns_g = lax.rsqrt(jnp.maximum(cg_ref[...], 1.0))
    yo_ref[...] = jnp.dot(xb, wo_ref[...],
                          preferred_element_type=jnp.float32) * ns_o
    yg_ref[...] = jnp.dot(xb, wg_ref[...],
                          preferred_element_type=jnp.float32) * ns_g


def _mm_call(x, Wo, Wg, cs_o, cs_g):
    blk = 1000
    return pl.pallas_call(
        _mm_body,
        grid=(N // blk,),
        in_specs=[
            pl.BlockSpec((blk, D), lambda i: (i, 0)),
            pl.BlockSpec((D, D), lambda i: (0, 0)),
            pl.BlockSpec((D, D), lambda i: (0, 0)),
            pl.BlockSpec((blk, 1), lambda i: (i, 0)),
            pl.BlockSpec((blk, 1), lambda i: (i, 0)),
        ],
        out_specs=[
            pl.BlockSpec((blk, D), lambda i: (i, 0)),
            pl.BlockSpec((blk, D), lambda i: (i, 0)),
        ],
        out_shape=[
            jax.ShapeDtypeStruct((N, D), jnp.float32),
            jax.ShapeDtypeStruct((N, D), jnp.float32),
        ],
    )(x, Wo, Wg, cs_o, cs_g)


def _main_call(y_o, y_g, ei_o, ei_g, z128):
    @functools.partial(
        pl.kernel,
        out_type=[
            jax.ShapeDtypeStruct((NT, RPT, D), jnp.float32),
            jax.ShapeDtypeStruct((NT, RPT, D), jnp.float32),
        ],
        mesh=_sc_mesh(),
        scratch_types=[
            pltpu.VMEM((2, KB, C), jnp.int32),
            pltpu.VMEM((2, KB, C), jnp.int32),
            pltpu.VMEM((2, C, D), jnp.float32),
            pltpu.VMEM_SHARED((NP, D), jnp.float32),
            pltpu.SemaphoreType.DMA,
            pltpu.SemaphoreType.DMA,
            pltpu.SemaphoreType.DMA,
            pltpu.SemaphoreType.DMA,
            pltpu.SemaphoreType.DMA,
            pltpu.SemaphoreType.DMA,
        ],
    )
    def main(y_o_hbm, y_g_hbm, ei_o_hbm, ei_g_hbm, z128_hbm,
             agg_o_out, agg_g_out, idx_src, idx_dst, rows, agg_sh,
             g0, g1, p0, p1, ss0, ss1):
        c = lax.axis_index("c")
        s = lax.axis_index("s")
        row0 = s * RPT
        pltpu.sync_copy(z128_hbm, agg_sh.at[pl.ds(row0, RPT)])
        plsc.subcore_barrier()

        gsems = (g0, g1)
        psems = (p0, p1)
        NBUF = 2
        LAG = 1

        def edge_loop(y_hbm, ei_hbm):
            pltpu.async_copy(ei_hbm.at[0, s, 0], idx_src.at[0], ss0)
            pltpu.async_copy(ei_hbm.at[1, s, 0], idx_dst.at[0], ss1)

            def b_body(b, carry):
                sl = b % 2
                pltpu.make_async_copy(ei_hbm.at[0, s, b], idx_src.at[sl],
                                      ss0).wait()
                pltpu.make_async_copy(ei_hbm.at[1, s, b], idx_dst.at[sl],
                                      ss1).wait()

                @pl.when(b + 1 < NBLK)
                def _():
                    pltpu.async_copy(ei_hbm.at[0, s, b + 1],
                                     idx_src.at[1 - sl], ss0)
                    pltpu.async_copy(ei_hbm.at[1, s, b + 1],
                                     idx_dst.at[1 - sl], ss1)

                dg = [None] * KB
                dsc = [None] * KB
                for t in range(KB + LAG):
                    jj = t
                    if jj < KB:
                        bb = jj % NBUF
                        if jj >= NBUF:
                            dsc[jj - NBUF].wait()
                        dg[jj] = pltpu.async_copy(
                            y_hbm.at[idx_src.at[sl, jj]], rows.at[bb],
                            gsems[bb])
                    kk = t - LAG
                    if 0 <= kk < KB:
                        bb = kk % NBUF
                        dg[kk].wait()
                        dsc[kk] = pltpu.async_copy(
                            rows.at[bb], agg_sh.at[idx_dst.at[sl, kk]],
                            psems[bb], add=True)
                for kk in range(KB - NBUF, KB):
                    dsc[kk].wait()
                return carry

            lax.fori_loop(0, NBLK, b_body, 0)

        @pl.when(c == 0)
        def _():
            edge_loop(y_o_hbm, ei_o_hbm)

        @pl.when(c == 1)
        def _():
            edge_loop(y_g_hbm, ei_g_hbm)

        plsc.subcore_barrier()

        @pl.when(c == 0)
        def _():
            pltpu.sync_copy(agg_sh.at[pl.ds(row0, RPT)], agg_o_out.at[s])

        @pl.when(c == 1)
        def _():
            pltpu.sync_copy(agg_sh.at[pl.ds(row0, RPT)], agg_g_out.at[s])

    return main(y_o, y_g, ei_o, ei_g, z128)


def _fin_body(ao_ref, ag_ref, cdo_ref, cdg_ref, bo_ref, bg_ref, out_ref):
    nd_o = lax.rsqrt(jnp.maximum(cdo_ref[...], 1.0))
    nd_g = lax.rsqrt(jnp.maximum(cdg_ref[...], 1.0))
    out_ref[...] = (ao_ref[...] * nd_o + ag_ref[...] * nd_g
                    + bo_ref[...] + bg_ref[...])


def _fin_call(agg_o, agg_g, cd_o, cd_g, bo, bg):
    blk = 1000
    return pl.pallas_call(
        _fin_body,
        grid=(N // blk,),
        in_specs=[
            pl.BlockSpec((blk, D), lambda i: (i, 0)),
            pl.BlockSpec((blk, D), lambda i: (i, 0)),
            pl.BlockSpec((blk, 1), lambda i: (i, 0)),
            pl.BlockSpec((blk, 1), lambda i: (i, 0)),
            pl.BlockSpec((1, D), lambda i: (0, 0)),
            pl.BlockSpec((1, D), lambda i: (0, 0)),
        ],
        out_specs=pl.BlockSpec((blk, D), lambda i: (i, 0)),
        out_shape=jax.ShapeDtypeStruct((N, D), jnp.float32),
    )(agg_o, agg_g, cd_o, cd_g, bo, bg)


def kernel(x, edge_index_orders, edge_index_geographical,
           W_orders, b_orders, W_geographical, b_geographical):
    ar = jnp.arange(EP - E, dtype=jnp.int32) % NPAD
    pad2 = jnp.stack([ar, ar + N])

    def prep(ei):
        return jnp.concatenate([ei, pad2], axis=1).reshape(2, NT, NBLK, KB, C)

    ei_o = prep(edge_index_orders)
    ei_g = prep(edge_index_geographical)
    ones1 = jnp.ones((C,), jnp.float32)
    z1 = jnp.zeros((WIN,), jnp.float32)
    z128 = jnp.zeros((RPT, D), jnp.float32)

    cnt = _deg_call(ei_o, ei_g, z1, ones1).reshape(2, 2, NPX)
    pc = jnp.zeros((N,), jnp.float32).at[:NPAD].set((EP - E) // NPAD)
    cs_o = (cnt[0, 0, :N] - pc)[:, None]
    cd_o = cnt[0, 1, :N, None]
    cs_g = (cnt[1, 0, :N] - pc)[:, None]
    cd_g = cnt[1, 1, :N, None]

    y_o, y_g = _mm_call(x, W_orders, W_geographical, cs_o, cs_g)
    agg_o, agg_g = _main_call(y_o, y_g, ei_o, ei_g, z128)
    return _fin_call(agg_o.reshape(NP, D), agg_g.reshape(NP, D), cd_o, cd_g,
                     b_orders.reshape(1, D), b_geographical.reshape(1, D))


# speedup vs baseline: 24.7849x; 1.0323x over previous
import functools

import jax
import jax.numpy as jnp
from jax import lax
from jax.experimental import pallas as pl
from jax.experimental.pallas import tpu as pltpu
from jax.experimental.pallas import tpu_sc as plsc

N = 10000
NPAD = 16
NP = N + NPAD
E = 320000
D = 128
NT = 16
C = 128
KB = 16
NBLK = 10
EP = NT * NBLK * KB * C
RPT = NP // NT


def _sc_mesh():
    return plsc.VectorSubcoreMesh(core_axis_name="c", subcore_axis_name="s")


WIN = 640
NPX = NT * WIN
WR = WIN // 128


def _deg_call(ei_o, ei_g, z1, ones1):
    @functools.partial(
        pl.kernel,
        out_type=jax.ShapeDtypeStruct((2, 2, NT, WR, 128), jnp.float32),
        mesh=_sc_mesh(),
        scratch_types=[
            pltpu.VMEM((2, KB, C), jnp.int32),
            pltpu.VMEM((2, KB, C), jnp.int32),
            pltpu.VMEM((C,), jnp.float32),
            pltpu.VMEM((WIN,), jnp.float32),
            pltpu.VMEM((WR, 128), jnp.float32),
            pltpu.VMEM_SHARED((NPX,), jnp.float32),
            pltpu.VMEM_SHARED((NPX,), jnp.float32),
            pltpu.SemaphoreType.DMA,
            pltpu.SemaphoreType.DMA,
            pltpu.SemaphoreType.DMA,
            pltpu.SemaphoreType.DMA,
            pltpu.SemaphoreType.DMA,
            pltpu.SemaphoreType.DMA,
        ],
    )
    def deg(ei_o_hbm, ei_g_hbm, z1_hbm, ones_hbm, cnt_out,
            idx_src, idx_dst, ones_v, buf1, buf2, cs_sh, cd_sh,
            sA0, sA1, sB0, sB1, ss0, ss1):
        sems = ((sA0, sA1), (sB0, sB1))
        c = lax.axis_index("c")
        s = lax.axis_index("s")
        w0 = s * WIN
        pltpu.sync_copy(ones_hbm, ones_v)
        pltpu.sync_copy(z1_hbm, cs_sh.at[pl.ds(w0, WIN)])
        pltpu.sync_copy(z1_hbm, cd_sh.at[pl.ds(w0, WIN)])
        plsc.subcore_barrier()

        def edge_loop(ei_hbm):
            pltpu.async_copy(ei_hbm.at[0, s, 0], idx_src.at[0], ss0)
            pltpu.async_copy(ei_hbm.at[1, s, 0], idx_dst.at[0], ss1)

            def b_body(b, carry):
                sl = b % 2
                pltpu.make_async_copy(ei_hbm.at[0, s, b], idx_src.at[sl],
                                      ss0).wait()
                pltpu.make_async_copy(ei_hbm.at[1, s, b], idx_dst.at[sl],
                                      ss1).wait()

                @pl.when(b + 1 < NBLK)
                def _():
                    pltpu.async_copy(ei_hbm.at[0, s, b + 1],
                                     idx_src.at[1 - sl], ss0)
                    pltpu.async_copy(ei_hbm.at[1, s, b + 1],
                                     idx_dst.at[1 - sl], ss1)

                prev = None
                for jj in range(KB):
                    cur = (
                        pltpu.async_copy(ones_v,
                                         cs_sh.at[idx_src.at[sl, jj]],
                                         sems[jj % 2][0], add=True),
                        pltpu.async_copy(ones_v,
                                         cd_sh.at[idx_dst.at[sl, jj]],
                                         sems[jj % 2][1], add=True),
                    )
                    if prev is not None:
                        prev[0].wait()
                        prev[1].wait()
                    prev = cur
                prev[0].wait()
                prev[1].wait()
                return carry

            lax.fori_loop(0, NBLK, b_body, 0)

        @pl.when(c == 0)
        def _():
            edge_loop(ei_o_hbm)

        @pl.when(c == 1)
        def _():
            edge_loop(ei_g_hbm)

        plsc.subcore_barrier()

        def emit(sh, m, side):
            pltpu.sync_copy(sh.at[pl.ds(w0, WIN)], buf1)
            for k in range(WIN // 16):
                buf2[k // 8, pl.ds((k % 8) * 16, 16)] = buf1[pl.ds(k * 16, 16)]
            pltpu.sync_copy(buf2, cnt_out.at[m, side, s])

        @pl.when(c == 0)
        def _():
            emit(cs_sh, 0, 0)
            emit(cd_sh, 0, 1)

        @pl.when(c == 1)
        def _():
            emit(cs_sh, 1, 0)
            emit(cd_sh, 1, 1)

    return deg(ei_o, ei_g, z1, ones1)


def _mm_body(x_ref, wo_ref, wg_ref, co_ref, cg_ref, yo_ref, yg_ref):
    xb = x_ref[...]
    ns_o = lax.rsqrt(jnp.maximum(co_ref[...], 1.0))
    ns_g = lax.rsqrt(jnp.maximum(cg_ref[...], 1.0))
    yo_ref[...] = jnp.dot(xb, wo_ref[...],
                          preferred_element_type=jnp.float32) * ns_o
    yg_ref[...] = jnp.dot(xb, wg_ref[...],
                          preferred_element_type=jnp.float32) * ns_g


def _mm_call(x, Wo, Wg, cs_o, cs_g):
    blk = 1000
    return pl.pallas_call(
        _mm_body,
        grid=(N // blk,),
        in_specs=[
            pl.BlockSpec((blk, D), lambda i: (i, 0)),
            pl.BlockSpec((D, D), lambda i: (0, 0)),
            pl.BlockSpec((D, D), lambda i: (0, 0)),
            pl.BlockSpec((blk, 1), lambda i: (i, 0)),
            pl.BlockSpec((blk, 1), lambda i: (i, 0)),
        ],
        out_specs=[
            pl.BlockSpec((blk, D), lambda i: (i, 0)),
            pl.BlockSpec((blk, D), lambda i: (i, 0)),
        ],
        out_shape=[
            jax.ShapeDtypeStruct((N, D), jnp.float32),
            jax.ShapeDtypeStruct((N, D), jnp.float32),
        ],
    )(x, Wo, Wg, cs_o, cs_g)


def _main_call(y_o, y_g, ei_o, ei_g, z128):
    @functools.partial(
        pl.kernel,
        out_type=[
            jax.ShapeDtypeStruct((NT, RPT, D), jnp.float32),
            jax.ShapeDtypeStruct((NT, RPT, D), jnp.float32),
        ],
        mesh=_sc_mesh(),
        scratch_types=[
            pltpu.VMEM((2, KB, C), jnp.int32),
            pltpu.VMEM((2, KB, C), jnp.int32),
            pltpu.VMEM((2, C, D), jnp.float32),
            pltpu.VMEM_SHARED((NP, D), jnp.float32),
            pltpu.SemaphoreType.DMA,
            pltpu.SemaphoreType.DMA,
            pltpu.SemaphoreType.DMA,
            pltpu.SemaphoreType.DMA,
            pltpu.SemaphoreType.DMA,
            pltpu.SemaphoreType.DMA,
        ],
    )
    def main(y_o_hbm, y_g_hbm, ei_o_hbm, ei_g_hbm, z128_hbm,
             agg_o_out, agg_g_out, idx_src, idx_dst, rows, agg_sh,
             g0, g1, p0, p1, ss0, ss1):
        c = lax.axis_index("c")
        s = lax.axis_index("s")
        row0 = s * RPT
        pltpu.sync_copy(z128_hbm, agg_sh.at[pl.ds(row0, RPT)])
        plsc.subcore_barrier()

        gsems = (g0, g1)
        psems = (p0, p1)
        NBUF = 2
        LAG = 1

        def edge_loop(y_hbm, ei_hbm):
            pltpu.async_copy(ei_hbm.at[0, s, 0], idx_src.at[0], ss0)
            pltpu.async_copy(ei_hbm.at[1, s, 0], idx_dst.at[0], ss1)

            def b_body(b, carry):
                sl = b % 2
                pltpu.make_async_copy(ei_hbm.at[0, s, b], idx_src.at[sl],
                                      ss0).wait()
                pltpu.make_async_copy(ei_hbm.at[1, s, b], idx_dst.at[sl],
                                      ss1).wait()

                @pl.when(b + 1 < NBLK)
                def _():
                    pltpu.async_copy(ei_hbm.at[0, s, b + 1],
                                     idx_src.at[1 - sl], ss0)
                    pltpu.async_copy(ei_hbm.at[1, s, b + 1],
                                     idx_dst.at[1 - sl], ss1)

                dg = [None] * KB
                dsc = [None] * KB
                for t in range(KB + LAG):
                    jj = t
                    if jj < KB:
                        bb = jj % NBUF
                        if jj >= NBUF:
                            dsc[jj - NBUF].wait()
                        dg[jj] = pltpu.async_copy(
                            y_hbm.at[idx_src.at[sl, jj]], rows.at[bb],
                            gsems[bb])
                    kk = t - LAG
                    if 0 <= kk < KB:
                        bb = kk % NBUF
                        dg[kk].wait()
                        dsc[kk] = pltpu.async_copy(
                            rows.at[bb], agg_sh.at[idx_dst.at[sl, kk]],
                            psems[bb], add=True)
                for kk in range(KB - NBUF, KB):
                    dsc[kk].wait()
                return carry

            lax.fori_loop(0, NBLK, b_body, 0)

        @pl.when(c == 0)
        def _():
            edge_loop(y_o_hbm, ei_o_hbm)

        @pl.when(c == 1)
        def _():
            edge_loop(y_g_hbm, ei_g_hbm)

        plsc.subcore_barrier()

        @pl.when(c == 0)
        def _():
            pltpu.sync_copy(agg_sh.at[pl.ds(row0, RPT)], agg_o_out.at[s])

        @pl.when(c == 1)
        def _():
            pltpu.sync_copy(agg_sh.at[pl.ds(row0, RPT)], agg_g_out.at[s])

    return main(y_o, y_g, ei_o, ei_g, z128)


def _fin_body(ao_ref, ag_ref, cdo_ref, cdg_ref, bo_ref, bg_ref, out_ref):
    nd_o = lax.rsqrt(jnp.maximum(cdo_ref[...], 1.0))
    nd_g = lax.rsqrt(jnp.maximum(cdg_ref[...], 1.0))
    out_ref[...] = (ao_ref[...] * nd_o + ag_ref[...] * nd_g
                    + bo_ref[...] + bg_ref[...])


def _fin_call(agg_o, agg_g, cd_o, cd_g, bo, bg):
    blk = 1000
    return pl.pallas_call(
        _fin_body,
        grid=(N // blk,),
        in_specs=[
            pl.BlockSpec((blk, D), lambda i: (i, 0)),
            pl.BlockSpec((blk, D), lambda i: (i, 0)),
            pl.BlockSpec((blk, 1), lambda i: (i, 0)),
            pl.BlockSpec((blk, 1), lambda i: (i, 0)),
            pl.BlockSpec((1, D), lambda i: (0, 0)),
            pl.BlockSpec((1, D), lambda i: (0, 0)),
        ],
        out_specs=pl.BlockSpec((blk, D), lambda i: (i, 0)),
        out_shape=jax.ShapeDtypeStruct((N, D), jnp.float32),
    )(agg_o, agg_g, cd_o, cd_g, bo, bg)


def kernel(x, edge_index_orders, edge_index_geographical,
           W_orders, b_orders, W_geographical, b_geographical):
    ar = jnp.arange(EP - E, dtype=jnp.int32) % NPAD
    pad2 = jnp.stack([ar, ar + N])

    def prep(ei):
        return jnp.concatenate([ei, pad2], axis=1).reshape(2, NT, NBLK, KB, C)

    ei_o = prep(edge_index_orders)
    ei_g = prep(edge_index_geographical)
    ones1 = jnp.ones((C,), jnp.float32)
    z1 = jnp.zeros((WIN,), jnp.float32)
    z128 = jnp.zeros((RPT, D), jnp.float32)

    cnt = _deg_call(ei_o, ei_g, z1, ones1).reshape(2, 2, NPX)
    pc = jnp.zeros((N,), jnp.float32).at[:NPAD].set((EP - E) // NPAD)
    cs_o = (cnt[0, 0, :N] - pc)[:, None]
    cd_o = cnt[0, 1, :N, None]
    cs_g = (cnt[1, 0, :N] - pc)[:, None]
    cd_g = cnt[1, 1, :N, None]

    y_o, y_g = _mm_call(x, W_orders, W_geographical, cs_o, cs_g)
    agg_o, agg_g = _main_call(y_o, y_g, ei_o, ei_g, z128)
    return _fin_call(agg_o.reshape(NP, D), agg_g.reshape(NP, D), cd_o, cd_g,
                     b_orders.reshape(1, D), b_geographical.reshape(1, D))
